# Initial kernel scaffold; baseline (speedup 1.0000x reference)
import jax
import jax.numpy as jnp
from jax.experimental import pallas as pl


def kernel(x, edge_index, W1, b1, W2, b2, W_cheb, Wf, bf, Wx, bx, Wc, bc):
    raise NotImplementedError("write your pallas kernel here")


# speedup vs baseline: 18.5515x; 18.5515x over previous
import functools

import jax
import jax.numpy as jnp
from jax import lax
from jax.experimental import pallas as pl
from jax.experimental.pallas import tpu as pltpu
from jax.experimental.pallas import tpu_sc as plsc

NC = 2
NS = 16
NW = NC * NS
CH = 128
BR = 10


def _mlp_body(x_ref, W1_ref, b1_ref, W2_ref, b2_ref, h_ref):
    t = jnp.maximum(
        jnp.dot(x_ref[...], W1_ref[...], preferred_element_type=jnp.float32)
        + b1_ref[...], 0.0)
    h_ref[...] = (
        jnp.dot(t, W2_ref[...], preferred_element_type=jnp.float32)
        + b2_ref[...])


def _mlp(x, W1, b1, W2, b2, blk):
    n, d = x.shape
    h_dim = W1.shape[1]
    return pl.pallas_call(
        _mlp_body,
        grid=(n // blk,),
        in_specs=[
            pl.BlockSpec((blk, d), lambda i: (i, 0)),
            pl.BlockSpec((d, h_dim), lambda i: (0, 0)),
            pl.BlockSpec((1, h_dim), lambda i: (0, 0)),
            pl.BlockSpec((h_dim, h_dim), lambda i: (0, 0)),
            pl.BlockSpec((1, h_dim), lambda i: (0, 0)),
        ],
        out_specs=pl.BlockSpec((blk, h_dim), lambda i: (i, 0)),
        out_shape=jax.ShapeDtypeStruct((n, h_dim), jnp.float32),
    )(x, W1, b1.reshape(1, -1), W2, b2.reshape(1, -1))


def _dinv_of(degp):
    deg = degp[0, :, 0:1] + degp[1, :, 0:1]
    return jnp.where(deg > 0,
                     lax.rsqrt(jnp.maximum(deg, 1e-12)),
                     jnp.zeros_like(deg))


def _scale_body(h_ref, degp_ref, g_ref):
    g_ref[...] = _dinv_of(degp_ref[...]) * h_ref[...]


def _scale(h, degp, blk):
    n, h_dim = h.shape
    return pl.pallas_call(
        _scale_body,
        grid=(n // blk,),
        in_specs=[
            pl.BlockSpec((blk, h_dim), lambda i: (i, 0)),
            pl.BlockSpec((NC, blk, h_dim), lambda i: (0, i, 0)),
        ],
        out_specs=pl.BlockSpec((blk, h_dim), lambda i: (i, 0)),
        out_shape=jax.ShapeDtypeStruct((n, h_dim), jnp.float32),
    )(h, degp)


def _deg_sc(src3, ones_rows, zrows, n_pad, h_dim):
    nbat = src3.shape[0]
    rpt = n_pad // NS
    mesh = plsc.VectorSubcoreMesh(core_axis_name="c", subcore_axis_name="s")

    @functools.partial(
        pl.kernel,
        out_type=jax.ShapeDtypeStruct((NC, n_pad, h_dim), jnp.float32),
        mesh=mesh,
        scratch_types=[
            pltpu.VMEM((BR, CH), jnp.int32),
            pltpu.VMEM((CH, h_dim), jnp.float32),
            pltpu.VMEM_SHARED((n_pad, h_dim), jnp.float32),
        ],
    )
    def deg_kernel(src_hbm, ones_hbm, z_hbm, out_hbm, idx_v, ones_v, acc_sh):
        c = lax.axis_index("c")
        s = lax.axis_index("s")
        w = s * NC + c
        pltpu.sync_copy(ones_hbm, ones_v)
        pltpu.sync_copy(z_hbm, acc_sh.at[pl.ds(s * rpt, rpt)])
        plsc.subcore_barrier()

        @pl.loop(w, nbat, step=NW)
        def _(b):
            pltpu.sync_copy(src_hbm.at[b], idx_v)
            for j in range(BR):
                pltpu.sync_copy(ones_v, acc_sh.at[idx_v.at[j]], add=True)

        plsc.subcore_barrier()
        pltpu.sync_copy(acc_sh.at[pl.ds(s * rpt, rpt)],
                        out_hbm.at[c, pl.ds(s * rpt, rpt)])

    return deg_kernel(src3, ones_rows, zrows)


def _prop_sc(src3, dst3, g, zrows, n_pad):
    h_dim = g.shape[1]
    nbat = src3.shape[0]
    rpt = n_pad // NS
    mesh = plsc.VectorSubcoreMesh(core_axis_name="c", subcore_axis_name="s")

    @functools.partial(
        pl.kernel,
        out_type=jax.ShapeDtypeStruct((NC, n_pad, h_dim), jnp.float32),
        mesh=mesh,
        scratch_types=[
            pltpu.VMEM((BR, CH), jnp.int32),
            pltpu.VMEM((BR, CH), jnp.int32),
            pltpu.VMEM((CH, h_dim), jnp.float32),
            pltpu.VMEM_SHARED((n_pad, h_dim), jnp.float32),
        ],
    )
    def prop_kernel(src_hbm, dst_hbm, g_hbm, z_hbm, out_hbm,
                    sidx_v, didx_v, rows_v, acc_sh):
        c = lax.axis_index("c")
        s = lax.axis_index("s")
        w = s * NC + c
        pltpu.sync_copy(z_hbm, acc_sh.at[pl.ds(s * rpt, rpt)])
        plsc.subcore_barrier()

        @pl.loop(w, nbat, step=NW)
        def _(b):
            pltpu.sync_copy(src_hbm.at[b], sidx_v)
            pltpu.sync_copy(dst_hbm.at[b], didx_v)
            for j in range(BR):
                pltpu.sync_copy(g_hbm.at[sidx_v.at[j]], rows_v)
                pltpu.sync_copy(rows_v, acc_sh.at[didx_v.at[j]], add=True)

        plsc.subcore_barrier()
        pltpu.sync_copy(acc_sh.at[pl.ds(s * rpt, rpt)],
                        out_hbm.at[c, pl.ds(s * rpt, rpt)])

    return prop_kernel(src3, dst3, g, zrows)


def _head_body(h_ref, p_ref, degp_ref, Wch_ref, Wf_ref, bf_ref,
               Wx_ref, bx_ref, Wc_ref, bc_ref, y_ref):
    h = h_ref[...]
    dinv = _dinv_of(degp_ref[...])
    p = p_ref[...]
    ph = -dinv * (p[0] + p[1])
    Wch = Wch_ref[...]
    dot = lambda a, b: jnp.dot(a, b, preferred_element_type=jnp.float32)
    f0 = dot(h, Wch[0, 0]) + dot(ph, Wch[0, 1])
    f1 = dot(h, Wch[1, 0]) + dot(ph, Wch[1, 1])
    Wf = Wf_ref[...]
    bf = bf_ref[...]
    hp0 = jnp.tanh(dot(f0, Wf) + bf)
    hp1 = jnp.tanh(dot(f1, Wf) + bf)
    xp = jnp.tanh(dot(h, Wx_ref[...]) + bx_ref[...])
    s0 = jnp.sum(hp0 * xp, axis=1, keepdims=True)
    s1 = jnp.sum(hp1 * xp, axis=1, keepdims=True)
    m = jnp.maximum(s0, s1)
    e0 = jnp.exp(s0 - m)
    e1 = jnp.exp(s1 - m)
    res = (f0 * e0 + f1 * e1) / (e0 + e1)
    y_ref[...] = dot(res, Wc_ref[...]) + bc_ref[...]


def _head(h, part, degp, W_cheb, Wf, bf, Wx, bx, Wc_pad, bc_pad, blk):
    n, h_dim = h.shape
    return pl.pallas_call(
        _head_body,
        grid=(n // blk,),
        in_specs=[
            pl.BlockSpec((blk, h_dim), lambda i: (i, 0)),
            pl.BlockSpec((NC, blk, h_dim), lambda i: (0, i, 0)),
            pl.BlockSpec((NC, blk, h_dim), lambda i: (0, i, 0)),
            pl.BlockSpec(W_cheb.shape, lambda i: (0, 0, 0, 0)),
            pl.BlockSpec((h_dim, h_dim), lambda i: (0, 0)),
            pl.BlockSpec((1, h_dim), lambda i: (0, 0)),
            pl.BlockSpec((h_dim, h_dim), lambda i: (0, 0)),
            pl.BlockSpec((1, h_dim), lambda i: (0, 0)),
            pl.BlockSpec((h_dim, h_dim), lambda i: (0, 0)),
            pl.BlockSpec((1, h_dim), lambda i: (0, 0)),
        ],
        out_specs=pl.BlockSpec((blk, h_dim), lambda i: (i, 0)),
        out_shape=jax.ShapeDtypeStruct((n, h_dim), jnp.float32),
    )(h, part, degp, W_cheb, Wf, bf.reshape(1, -1),
      Wx, bx.reshape(1, -1), Wc_pad, bc_pad)


def kernel(x, edge_index, W1, b1, W2, b2, W_cheb, Wf, bf, Wx, bx, Wc, bc):
    n, _ = x.shape
    e = edge_index.shape[1]
    h_dim = W1.shape[1]
    c_dim = Wc.shape[1]
    nbat = e // (BR * CH)
    n_pad = ((n + 8 * NS - 1) // (8 * NS)) * (8 * NS)
    rpt = n_pad // NS
    blk = 1000

    src3 = edge_index[0].reshape(nbat, BR, CH)
    dst3 = edge_index[1].reshape(nbat, BR, CH)
    ones_rows = jnp.ones((CH, h_dim), jnp.float32)
    z128 = jnp.zeros((rpt, h_dim), jnp.float32)
    Wc_pad = jnp.zeros((h_dim, h_dim), jnp.float32).at[:, :c_dim].set(Wc)
    bc_pad = jnp.zeros((1, h_dim), jnp.float32).at[:, :c_dim].set(bc)

    h = _mlp(x, W1, b1, W2, b2, blk)
    degp = _deg_sc(src3, ones_rows, z128, n_pad, h_dim)
    g = _scale(h, degp, blk)
    part = _prop_sc(src3, dst3, g, z128, n_pad)
    y_pad = _head(h, part, degp, W_cheb, Wf, bf, Wx, bx, Wc_pad, bc_pad, blk)
    return y_pad[:, :c_dim]


# speedup vs baseline: 27.4656x; 1.4805x over previous
import dataclasses
import functools

import jax
import jax.numpy as jnp
from jax import lax
from jax.experimental import pallas as pl
from jax.experimental.pallas import tpu as pltpu
from jax.experimental.pallas import tpu_sc as plsc

NC = 2
NS = 16
NW = NC * NS
CH = 128
BR = 10


def _mlp_body(x_ref, W1_ref, b1_ref, W2_ref, b2_ref, h_ref):
    t = jnp.maximum(
        jnp.dot(x_ref[...], W1_ref[...], preferred_element_type=jnp.float32)
        + b1_ref[...], 0.0)
    h_ref[...] = (
        jnp.dot(t, W2_ref[...], preferred_element_type=jnp.float32)
        + b2_ref[...])


def _mlp(x, W1, b1, W2, b2, blk):
    n, d = x.shape
    h_dim = W1.shape[1]
    return pl.pallas_call(
        _mlp_body,
        grid=(n // blk,),
        in_specs=[
            pl.BlockSpec((blk, d), lambda i: (i, 0)),
            pl.BlockSpec((d, h_dim), lambda i: (0, 0)),
            pl.BlockSpec((1, h_dim), lambda i: (0, 0)),
            pl.BlockSpec((h_dim, h_dim), lambda i: (0, 0)),
            pl.BlockSpec((1, h_dim), lambda i: (0, 0)),
        ],
        out_specs=pl.BlockSpec((blk, h_dim), lambda i: (i, 0)),
        out_shape=jax.ShapeDtypeStruct((n, h_dim), jnp.float32),
    )(x, W1, b1.reshape(1, -1), W2, b2.reshape(1, -1))


def _dinv_of(degp):
    deg = degp[0] + degp[1]
    return jnp.where(deg > 0,
                     lax.rsqrt(jnp.maximum(deg, 1e-12)),
                     jnp.zeros_like(deg))


def _scale_body(h_ref, degp_ref, g_ref):
    g_ref[...] = _dinv_of(degp_ref[...]) * h_ref[...]


def _scale(h, degp, blk):
    n, h_dim = h.shape
    return pl.pallas_call(
        _scale_body,
        grid=(n // blk,),
        in_specs=[
            pl.BlockSpec((blk, h_dim), lambda i: (i, 0)),
            pl.BlockSpec((NC, blk, 1), lambda i: (0, i, 0)),
        ],
        out_specs=pl.BlockSpec((blk, h_dim), lambda i: (i, 0)),
        out_shape=jax.ShapeDtypeStruct((n, h_dim), jnp.float32),
    )(h, degp)


def _deg_sc(src_flat, n_pad):
    e = src_flat.shape[0]
    bsz = 1280
    nbat = e // bsz
    rpt = n_pad // NS
    mesh = plsc.VectorSubcoreMesh(core_axis_name="c", subcore_axis_name="s")
    cp = dataclasses.replace(pltpu.CompilerParams(),
                             needs_layout_passes=False)

    @functools.partial(
        pl.kernel,
        out_type=jax.ShapeDtypeStruct((NC * n_pad,), jnp.float32),
        mesh=mesh,
        compiler_params=cp,
        scratch_types=[
            pltpu.VMEM((bsz,), jnp.int32),
            pltpu.VMEM((n_pad,), jnp.float32),
            pltpu.VMEM((n_pad,), jnp.float32),
            pltpu.VMEM((rpt,), jnp.float32),
            pltpu.VMEM_SHARED((NS * n_pad,), jnp.float32),
            pltpu.SemaphoreType.DMA,
        ],
    )
    def deg_kernel(src_hbm, out_hbm, idx_v, hist_v, load_v, out_v,
                   stage_sh, sem):
        c = lax.axis_index("c")
        s = lax.axis_index("s")
        w = s * NC + c

        @pl.loop(0, n_pad, step=16)
        def _(i):
            hist_v[pl.ds(i, 16)] = jnp.zeros((16,), jnp.float32)

        ones16 = jnp.ones((16,), jnp.float32)

        @pl.loop(w, nbat, step=NW)
        def _(b):
            pltpu.sync_copy(src_hbm.at[pl.ds(b * bsz, bsz)], idx_v)
            for k in range(bsz // 16):
                plsc.addupdate_scatter(
                    hist_v, [idx_v[pl.ds(k * 16, 16)]], ones16)

        pltpu.sync_copy(hist_v, stage_sh.at[pl.ds(s * n_pad, n_pad)])
        plsc.subcore_barrier()
        descs = [
            pltpu.async_copy(stage_sh.at[pl.ds(r * n_pad + s * rpt, rpt)],
                             load_v.at[pl.ds(r * rpt, rpt)], sem)
            for r in range(NS)
        ]
        for dsc in descs:
            dsc.wait()

        @pl.loop(0, rpt, step=16)
        def _(k):
            t = load_v[pl.ds(k, 16)]
            for r in range(1, NS):
                t = t + load_v[pl.ds(r * rpt + k, 16)]
            out_v[pl.ds(k, 16)] = t

        pltpu.sync_copy(out_v, out_hbm.at[pl.ds(c * n_pad + s * rpt, rpt)])

    return deg_kernel(src_flat)


def _prop_sc(src3, dst3, g, zrows, n_pad):
    h_dim = g.shape[1]
    nbat = src3.shape[0]
    rpt = n_pad // NS
    mesh = plsc.VectorSubcoreMesh(core_axis_name="c", subcore_axis_name="s")

    @functools.partial(
        pl.kernel,
        out_type=jax.ShapeDtypeStruct((NC, n_pad, h_dim), jnp.float32),
        mesh=mesh,
        scratch_types=[
            pltpu.VMEM((BR, CH), jnp.int32),
            pltpu.VMEM((BR, CH), jnp.int32),
        ] + [pltpu.VMEM((CH, h_dim), jnp.float32)] * 2
          + [pltpu.VMEM_SHARED((n_pad, h_dim), jnp.float32)]
          + [pltpu.SemaphoreType.DMA] * 4,
    )
    def prop_kernel(src_hbm, dst_hbm, g_hbm, z_hbm, out_hbm,
                    sidx_v, didx_v, r0, r1, acc_sh,
                    g0, g1, s0, s1):
        rows = (r0, r1)
        gsem = (g0, g1)
        ssem = (s0, s1)
        nbuf = 2
        c = lax.axis_index("c")
        s = lax.axis_index("s")
        w = s * NC + c
        pltpu.sync_copy(z_hbm, acc_sh.at[pl.ds(s * rpt, rpt)])
        plsc.subcore_barrier()

        @pl.loop(w, nbat, step=NW)
        def _(b):
            pltpu.sync_copy(src_hbm.at[b], sidx_v)
            pltpu.sync_copy(dst_hbm.at[b], didx_v)
            gat = [None] * BR
            scat = [None] * BR
            for j in range(BR):
                bi = j % nbuf
                if j >= nbuf:
                    scat[j - nbuf].wait()
                gat[j] = pltpu.async_copy(
                    g_hbm.at[sidx_v.at[j]], rows[bi], gsem[bi])
                if j >= 1:
                    k = j - 1
                    gat[k].wait()
                    scat[k] = pltpu.async_copy(
                        rows[k % nbuf], acc_sh.at[didx_v.at[k]],
                        ssem[k % nbuf], add=True)
            k = BR - 1
            gat[k].wait()
            scat[k] = pltpu.async_copy(
                rows[k % nbuf], acc_sh.at[didx_v.at[k]],
                ssem[k % nbuf], add=True)
            for k in range(BR - nbuf, BR):
                scat[k].wait()

        plsc.subcore_barrier()
        pltpu.sync_copy(acc_sh.at[pl.ds(s * rpt, rpt)],
                        out_hbm.at[c, pl.ds(s * rpt, rpt)])

    return prop_kernel(src3, dst3, g, zrows)


def _head_body(h_ref, p_ref, degp_ref, Wch_ref, Wf_ref, bf_ref,
               Wx_ref, bx_ref, Wc_ref, bc_ref, y_ref):
    h = h_ref[...]
    dinv = _dinv_of(degp_ref[...])
    p = p_ref[...]
    ph = -dinv * (p[0] + p[1])
    Wch = Wch_ref[...]
    dot = lambda a, b: jnp.dot(a, b, preferred_element_type=jnp.float32)
    f0 = dot(h, Wch[0, 0]) + dot(ph, Wch[0, 1])
    f1 = dot(h, Wch[1, 0]) + dot(ph, Wch[1, 1])
    Wf = Wf_ref[...]
    bf = bf_ref[...]
    hp0 = jnp.tanh(dot(f0, Wf) + bf)
    hp1 = jnp.tanh(dot(f1, Wf) + bf)
    xp = jnp.tanh(dot(h, Wx_ref[...]) + bx_ref[...])
    s0 = jnp.sum(hp0 * xp, axis=1, keepdims=True)
    s1 = jnp.sum(hp1 * xp, axis=1, keepdims=True)
    m = jnp.maximum(s0, s1)
    e0 = jnp.exp(s0 - m)
    e1 = jnp.exp(s1 - m)
    res = (f0 * e0 + f1 * e1) / (e0 + e1)
    y_ref[...] = dot(res, Wc_ref[...]) + bc_ref[...]


def _head(h, part, degp, W_cheb, Wf, bf, Wx, bx, Wc_pad, bc_pad, blk):
    n, h_dim = h.shape
    return pl.pallas_call(
        _head_body,
        grid=(n // blk,),
        in_specs=[
            pl.BlockSpec((blk, h_dim), lambda i: (i, 0)),
            pl.BlockSpec((NC, blk, h_dim), lambda i: (0, i, 0)),
            pl.BlockSpec((NC, blk, 1), lambda i: (0, i, 0)),
            pl.BlockSpec(W_cheb.shape, lambda i: (0, 0, 0, 0)),
            pl.BlockSpec((h_dim, h_dim), lambda i: (0, 0)),
            pl.BlockSpec((1, h_dim), lambda i: (0, 0)),
            pl.BlockSpec((h_dim, h_dim), lambda i: (0, 0)),
            pl.BlockSpec((1, h_dim), lambda i: (0, 0)),
            pl.BlockSpec((h_dim, h_dim), lambda i: (0, 0)),
            pl.BlockSpec((1, h_dim), lambda i: (0, 0)),
        ],
        out_specs=pl.BlockSpec((blk, h_dim), lambda i: (i, 0)),
        out_shape=jax.ShapeDtypeStruct((n, h_dim), jnp.float32),
    )(h, part, degp, W_cheb, Wf, bf.reshape(1, -1),
      Wx, bx.reshape(1, -1), Wc_pad, bc_pad)


def kernel(x, edge_index, W1, b1, W2, b2, W_cheb, Wf, bf, Wx, bx, Wc, bc):
    n, _ = x.shape
    e = edge_index.shape[1]
    h_dim = W1.shape[1]
    c_dim = Wc.shape[1]
    nbat = e // (BR * CH)
    blk = 1024
    n_pad = ((n + blk - 1) // blk) * blk
    rpt = n_pad // NS

    src3 = edge_index[0].reshape(nbat, BR, CH)
    dst3 = edge_index[1].reshape(nbat, BR, CH)
    z128 = jnp.zeros((rpt, h_dim), jnp.float32)
    Wc_pad = jnp.zeros((h_dim, h_dim), jnp.float32).at[:, :c_dim].set(Wc)
    bc_pad = jnp.zeros((1, h_dim), jnp.float32).at[:, :c_dim].set(bc)
    x_pad = jnp.zeros((n_pad, x.shape[1]), jnp.float32).at[:n].set(x)

    h = _mlp(x_pad, W1, b1, W2, b2, blk)
    deg_flat = _deg_sc(edge_index[0], n_pad)
    degp = deg_flat.reshape(NC, n_pad, 1)
    g = _scale(h, degp, blk)
    part = _prop_sc(src3, dst3, g, z128, n_pad)
    y_pad = _head(h, part, degp, W_cheb, Wf, bf, Wx, bx, Wc_pad, bc_pad, blk)
    return y_pad[:n, :c_dim]


# speedup vs baseline: 28.4674x; 1.0365x over previous
import dataclasses
import functools

import jax
import jax.numpy as jnp
from jax import lax
from jax.experimental import pallas as pl
from jax.experimental.pallas import tpu as pltpu
from jax.experimental.pallas import tpu_sc as plsc

NC = 2
NS = 16
NW = NC * NS
CH = 80
BR = 16


def _mlp_body(x_ref, W1_ref, b1_ref, W2_ref, b2_ref, h_ref):
    t = jnp.maximum(
        jnp.dot(x_ref[...], W1_ref[...], preferred_element_type=jnp.float32)
        + b1_ref[...], 0.0)
    h_ref[...] = (
        jnp.dot(t, W2_ref[...], preferred_element_type=jnp.float32)
        + b2_ref[...])


def _mlp(x, W1, b1, W2, b2, blk):
    n, d = x.shape
    h_dim = W1.shape[1]
    return pl.pallas_call(
        _mlp_body,
        grid=(n // blk,),
        in_specs=[
            pl.BlockSpec((blk, d), lambda i: (i, 0)),
            pl.BlockSpec((d, h_dim), lambda i: (0, 0)),
            pl.BlockSpec((1, h_dim), lambda i: (0, 0)),
            pl.BlockSpec((h_dim, h_dim), lambda i: (0, 0)),
            pl.BlockSpec((1, h_dim), lambda i: (0, 0)),
        ],
        out_specs=pl.BlockSpec((blk, h_dim), lambda i: (i, 0)),
        out_shape=jax.ShapeDtypeStruct((n, h_dim), jnp.float32),
    )(x, W1, b1.reshape(1, -1), W2, b2.reshape(1, -1))


def _dinv_of(degp):
    deg = degp[0] + degp[1]
    return jnp.where(deg > 0,
                     lax.rsqrt(jnp.maximum(deg, 1e-12)),
                     jnp.zeros_like(deg))


def _scale_body(h_ref, degp_ref, g_ref):
    g_ref[...] = _dinv_of(degp_ref[...]) * h_ref[...]


def _scale(h, degp, blk):
    n, h_dim = h.shape
    return pl.pallas_call(
        _scale_body,
        grid=(n // blk,),
        in_specs=[
            pl.BlockSpec((blk, h_dim), lambda i: (i, 0)),
            pl.BlockSpec((NC, blk, 1), lambda i: (0, i, 0)),
        ],
        out_specs=pl.BlockSpec((blk, h_dim), lambda i: (i, 0)),
        out_shape=jax.ShapeDtypeStruct((n, h_dim), jnp.float32),
    )(h, degp)


def _deg_sc(src_flat, n_pad):
    e = src_flat.shape[0]
    bsz = 1280
    nbat = e // bsz
    rpt = n_pad // NS
    mesh = plsc.VectorSubcoreMesh(core_axis_name="c", subcore_axis_name="s")
    cp = dataclasses.replace(pltpu.CompilerParams(),
                             needs_layout_passes=False)

    @functools.partial(
        pl.kernel,
        out_type=jax.ShapeDtypeStruct((NC * n_pad,), jnp.float32),
        mesh=mesh,
        compiler_params=cp,
        scratch_types=[
            pltpu.VMEM((bsz,), jnp.int32),
            pltpu.VMEM((n_pad,), jnp.float32),
            pltpu.VMEM((n_pad,), jnp.float32),
            pltpu.VMEM((rpt,), jnp.float32),
            pltpu.VMEM_SHARED((NS * n_pad,), jnp.float32),
            pltpu.SemaphoreType.DMA,
        ],
    )
    def deg_kernel(src_hbm, out_hbm, idx_v, hist_v, load_v, out_v,
                   stage_sh, sem):
        c = lax.axis_index("c")
        s = lax.axis_index("s")
        w = s * NC + c

        @pl.loop(0, n_pad, step=16)
        def _(i):
            hist_v[pl.ds(i, 16)] = jnp.zeros((16,), jnp.float32)

        ones16 = jnp.ones((16,), jnp.float32)

        @pl.loop(w, nbat, step=NW)
        def _(b):
            pltpu.sync_copy(src_hbm.at[pl.ds(b * bsz, bsz)], idx_v)
            for k in range(bsz // 16):
                plsc.addupdate_scatter(
                    hist_v, [idx_v[pl.ds(k * 16, 16)]], ones16)

        pltpu.sync_copy(hist_v, stage_sh.at[pl.ds(s * n_pad, n_pad)])
        plsc.subcore_barrier()
        descs = [
            pltpu.async_copy(stage_sh.at[pl.ds(r * n_pad + s * rpt, rpt)],
                             load_v.at[pl.ds(r * rpt, rpt)], sem)
            for r in range(NS)
        ]
        for dsc in descs:
            dsc.wait()

        @pl.loop(0, rpt, step=16)
        def _(k):
            t = load_v[pl.ds(k, 16)]
            for r in range(1, NS):
                t = t + load_v[pl.ds(r * rpt + k, 16)]
            out_v[pl.ds(k, 16)] = t

        pltpu.sync_copy(out_v, out_hbm.at[pl.ds(c * n_pad + s * rpt, rpt)])

    return deg_kernel(src_flat)


def _prop_sc(src3, dst3, g, zrows, n_pad):
    h_dim = g.shape[1]
    nbat = src3.shape[0]
    rpt = n_pad // NS
    mesh = plsc.VectorSubcoreMesh(core_axis_name="c", subcore_axis_name="s")

    @functools.partial(
        pl.kernel,
        out_type=jax.ShapeDtypeStruct((NC, n_pad, h_dim), jnp.float32),
        mesh=mesh,
        scratch_types=[
            pltpu.VMEM((BR, CH), jnp.int32),
            pltpu.VMEM((BR, CH), jnp.int32),
        ] + [pltpu.VMEM((CH, h_dim), jnp.float32)] * 4
          + [pltpu.VMEM_SHARED((n_pad, h_dim), jnp.float32)]
          + [pltpu.SemaphoreType.DMA] * 9,
    )
    def prop_kernel(src_hbm, dst_hbm, g_hbm, z_hbm, out_hbm,
                    sidx_v, didx_v, r0, r1, r2, r3, acc_sh,
                    g0, g1, g2, g3, s0, s1, s2, s3, isem):
        rows = (r0, r1, r2, r3)
        gsem = (g0, g1, g2, g3)
        ssem = (s0, s1, s2, s3)
        nbuf = 4
        c = lax.axis_index("c")
        s = lax.axis_index("s")
        w = s * NC + c
        pltpu.sync_copy(z_hbm, acc_sh.at[pl.ds(s * rpt, rpt)])
        plsc.subcore_barrier()

        @pl.loop(w, nbat, step=NW)
        def _(b):
            i0 = pltpu.async_copy(src_hbm.at[b], sidx_v, isem)
            i1 = pltpu.async_copy(dst_hbm.at[b], didx_v, isem)
            i0.wait()
            i1.wait()
            gat = [None] * BR
            scat = [None] * BR
            for j in range(BR):
                bi = j % nbuf
                if j >= nbuf:
                    scat[j - nbuf].wait()
                gat[j] = pltpu.async_copy(
                    g_hbm.at[sidx_v.at[j]], rows[bi], gsem[bi])
                if j >= 2:
                    k = j - 2
                    gat[k].wait()
                    scat[k] = pltpu.async_copy(
                        rows[k % nbuf], acc_sh.at[didx_v.at[k]],
                        ssem[k % nbuf], add=True)
            for k in (BR - 2, BR - 1):
                gat[k].wait()
                scat[k] = pltpu.async_copy(
                    rows[k % nbuf], acc_sh.at[didx_v.at[k]],
                    ssem[k % nbuf], add=True)
            for k in range(BR - nbuf, BR):
                scat[k].wait()

        plsc.subcore_barrier()
        pltpu.sync_copy(acc_sh.at[pl.ds(s * rpt, rpt)],
                        out_hbm.at[c, pl.ds(s * rpt, rpt)])

    return prop_kernel(src3, dst3, g, zrows)


def _head_body(h_ref, p_ref, degp_ref, Wch_ref, Wf_ref, bf_ref,
               Wx_ref, bx_ref, Wc_ref, bc_ref, y_ref):
    h = h_ref[...]
    dinv = _dinv_of(degp_ref[...])
    p = p_ref[...]
    ph = -dinv * (p[0] + p[1])
    Wch = Wch_ref[...]
    dot = lambda a, b: jnp.dot(a, b, preferred_element_type=jnp.float32)
    f0 = dot(h, Wch[0, 0]) + dot(ph, Wch[0, 1])
    f1 = dot(h, Wch[1, 0]) + dot(ph, Wch[1, 1])
    Wf = Wf_ref[...]
    bf = bf_ref[...]
    hp0 = jnp.tanh(dot(f0, Wf) + bf)
    hp1 = jnp.tanh(dot(f1, Wf) + bf)
    xp = jnp.tanh(dot(h, Wx_ref[...]) + bx_ref[...])
    s0 = jnp.sum(hp0 * xp, axis=1, keepdims=True)
    s1 = jnp.sum(hp1 * xp, axis=1, keepdims=True)
    m = jnp.maximum(s0, s1)
    e0 = jnp.exp(s0 - m)
    e1 = jnp.exp(s1 - m)
    res = (f0 * e0 + f1 * e1) / (e0 + e1)
    y_ref[...] = dot(res, Wc_ref[...]) + bc_ref[...]


def _head(h, part, degp, W_cheb, Wf, bf, Wx, bx, Wc_pad, bc_pad, blk):
    n, h_dim = h.shape
    return pl.pallas_call(
        _head_body,
        grid=(n // blk,),
        in_specs=[
            pl.BlockSpec((blk, h_dim), lambda i: (i, 0)),
            pl.BlockSpec((NC, blk, h_dim), lambda i: (0, i, 0)),
            pl.BlockSpec((NC, blk, 1), lambda i: (0, i, 0)),
            pl.BlockSpec(W_cheb.shape, lambda i: (0, 0, 0, 0)),
            pl.BlockSpec((h_dim, h_dim), lambda i: (0, 0)),
            pl.BlockSpec((1, h_dim), lambda i: (0, 0)),
            pl.BlockSpec((h_dim, h_dim), lambda i: (0, 0)),
            pl.BlockSpec((1, h_dim), lambda i: (0, 0)),
            pl.BlockSpec((h_dim, h_dim), lambda i: (0, 0)),
            pl.BlockSpec((1, h_dim), lambda i: (0, 0)),
        ],
        out_specs=pl.BlockSpec((blk, h_dim), lambda i: (i, 0)),
        out_shape=jax.ShapeDtypeStruct((n, h_dim), jnp.float32),
    )(h, part, degp, W_cheb, Wf, bf.reshape(1, -1),
      Wx, bx.reshape(1, -1), Wc_pad, bc_pad)


def kernel(x, edge_index, W1, b1, W2, b2, W_cheb, Wf, bf, Wx, bx, Wc, bc):
    n, _ = x.shape
    e = edge_index.shape[1]
    h_dim = W1.shape[1]
    c_dim = Wc.shape[1]
    nbat = e // (BR * CH)
    blk = 1024
    n_pad = ((n + blk - 1) // blk) * blk
    rpt = n_pad // NS

    src3 = edge_index[0].reshape(nbat, BR, CH)
    dst3 = edge_index[1].reshape(nbat, BR, CH)
    z128 = jnp.zeros((rpt, h_dim), jnp.float32)
    Wc_pad = jnp.zeros((h_dim, h_dim), jnp.float32).at[:, :c_dim].set(Wc)
    bc_pad = jnp.zeros((1, h_dim), jnp.float32).at[:, :c_dim].set(bc)
    x_pad = jnp.zeros((n_pad, x.shape[1]), jnp.float32).at[:n].set(x)

    h = _mlp(x_pad, W1, b1, W2, b2, blk)
    deg_flat = _deg_sc(edge_index[0], n_pad)
    degp = deg_flat.reshape(NC, n_pad, 1)
    g = _scale(h, degp, blk)
    part = _prop_sc(src3, dst3, g, z128, n_pad)
    y_pad = _head(h, part, degp, W_cheb, Wf, bf, Wx, bx, Wc_pad, bc_pad, blk)
    return y_pad[:n, :c_dim]


# speedup vs baseline: 28.5999x; 1.0047x over previous
import dataclasses
import functools

import jax
import jax.numpy as jnp
from jax import lax
from jax.experimental import pallas as pl
from jax.experimental.pallas import tpu as pltpu
from jax.experimental.pallas import tpu_sc as plsc

NC = 2
NS = 16
NW = NC * NS
CH = 80
BR = 16


def _dinv_of(degp):
    deg = degp[0] + degp[1]
    return jnp.where(deg > 0,
                     lax.rsqrt(jnp.maximum(deg, 1e-12)),
                     jnp.zeros_like(deg))


def _mlpscale_body(x_ref, W1_ref, b1_ref, W2_ref, b2_ref, degp_ref,
                   h_ref, g_ref):
    t = jnp.maximum(
        jnp.dot(x_ref[...], W1_ref[...], preferred_element_type=jnp.float32)
        + b1_ref[...], 0.0)
    h = (jnp.dot(t, W2_ref[...], preferred_element_type=jnp.float32)
         + b2_ref[...])
    h_ref[...] = h
    g_ref[...] = _dinv_of(degp_ref[...]) * h


def _mlpscale(x, W1, b1, W2, b2, degp, blk):
    n, d = x.shape
    h_dim = W1.shape[1]
    out = jax.ShapeDtypeStruct((n, h_dim), jnp.float32)
    return pl.pallas_call(
        _mlpscale_body,
        grid=(n // blk,),
        in_specs=[
            pl.BlockSpec((blk, d), lambda i: (i, 0)),
            pl.BlockSpec((d, h_dim), lambda i: (0, 0)),
            pl.BlockSpec((1, h_dim), lambda i: (0, 0)),
            pl.BlockSpec((h_dim, h_dim), lambda i: (0, 0)),
            pl.BlockSpec((1, h_dim), lambda i: (0, 0)),
            pl.BlockSpec((NC, blk, 1), lambda i: (0, i, 0)),
        ],
        out_specs=[pl.BlockSpec((blk, h_dim), lambda i: (i, 0)),
                   pl.BlockSpec((blk, h_dim), lambda i: (i, 0))],
        out_shape=[out, out],
    )(x, W1, b1.reshape(1, -1), W2, b2.reshape(1, -1), degp)


def _deg_sc(src_flat, n_pad):
    e = src_flat.shape[0]
    bsz = 1280
    nbat = e // bsz
    rpt = n_pad // NS
    mesh = plsc.VectorSubcoreMesh(core_axis_name="c", subcore_axis_name="s")
    cp = dataclasses.replace(pltpu.CompilerParams(),
                             needs_layout_passes=False)

    @functools.partial(
        pl.kernel,
        out_type=jax.ShapeDtypeStruct((NC * n_pad,), jnp.float32),
        mesh=mesh,
        compiler_params=cp,
        scratch_types=[
            pltpu.VMEM((bsz,), jnp.int32),
            pltpu.VMEM((n_pad,), jnp.float32),
            pltpu.VMEM((n_pad,), jnp.float32),
            pltpu.VMEM((rpt,), jnp.float32),
            pltpu.VMEM_SHARED((NS * n_pad,), jnp.float32),
            pltpu.SemaphoreType.DMA,
        ],
    )
    def deg_kernel(src_hbm, out_hbm, idx_v, hist_v, load_v, out_v,
                   stage_sh, sem):
        c = lax.axis_index("c")
        s = lax.axis_index("s")
        w = s * NC + c

        @pl.loop(0, n_pad, step=16)
        def _(i):
            hist_v[pl.ds(i, 16)] = jnp.zeros((16,), jnp.float32)

        ones16 = jnp.ones((16,), jnp.float32)

        @pl.loop(w, nbat, step=NW)
        def _(b):
            pltpu.sync_copy(src_hbm.at[pl.ds(b * bsz, bsz)], idx_v)
            for k in range(bsz // 16):
                plsc.addupdate_scatter(
                    hist_v, [idx_v[pl.ds(k * 16, 16)]], ones16)

        pltpu.sync_copy(hist_v, stage_sh.at[pl.ds(s * n_pad, n_pad)])
        plsc.subcore_barrier()
        descs = [
            pltpu.async_copy(stage_sh.at[pl.ds(r * n_pad + s * rpt, rpt)],
                             load_v.at[pl.ds(r * rpt, rpt)], sem)
            for r in range(NS)
        ]
        for dsc in descs:
            dsc.wait()

        @pl.loop(0, rpt, step=16)
        def _(k):
            t = load_v[pl.ds(k, 16)]
            for r in range(1, NS):
                t = t + load_v[pl.ds(r * rpt + k, 16)]
            out_v[pl.ds(k, 16)] = t

        pltpu.sync_copy(out_v, out_hbm.at[pl.ds(c * n_pad + s * rpt, rpt)])

    return deg_kernel(src_flat)


def _prop_sc(src3, dst3, g, zrows, n_pad):
    h_dim = g.shape[1]
    nbat = src3.shape[0]
    rpt = n_pad // NS
    mesh = plsc.VectorSubcoreMesh(core_axis_name="c", subcore_axis_name="s")

    @functools.partial(
        pl.kernel,
        out_type=jax.ShapeDtypeStruct((NC, n_pad, h_dim), jnp.float32),
        mesh=mesh,
        scratch_types=[
            pltpu.VMEM((BR, CH), jnp.int32),
            pltpu.VMEM((BR, CH), jnp.int32),
        ] + [pltpu.VMEM((CH, h_dim), jnp.float32)] * 4
          + [pltpu.VMEM_SHARED((n_pad, h_dim), jnp.float32)]
          + [pltpu.SemaphoreType.DMA] * 9,
    )
    def prop_kernel(src_hbm, dst_hbm, g_hbm, z_hbm, out_hbm,
                    sidx_v, didx_v, r0, r1, r2, r3, acc_sh,
                    g0, g1, g2, g3, s0, s1, s2, s3, isem):
        rows = (r0, r1, r2, r3)
        gsem = (g0, g1, g2, g3)
        ssem = (s0, s1, s2, s3)
        nbuf = 4
        c = lax.axis_index("c")
        s = lax.axis_index("s")
        w = s * NC + c
        pltpu.sync_copy(z_hbm, acc_sh.at[pl.ds(s * rpt, rpt)])
        plsc.subcore_barrier()

        @pl.loop(w, nbat, step=NW)
        def _(b):
            i0 = pltpu.async_copy(src_hbm.at[b], sidx_v, isem)
            i1 = pltpu.async_copy(dst_hbm.at[b], didx_v, isem)
            i0.wait()
            i1.wait()
            gat = [None] * BR
            scat = [None] * BR
            for j in range(BR):
                bi = j % nbuf
                if j >= nbuf:
                    scat[j - nbuf].wait()
                gat[j] = pltpu.async_copy(
                    g_hbm.at[sidx_v.at[j]], rows[bi], gsem[bi])
                if j >= 2:
                    k = j - 2
                    gat[k].wait()
                    scat[k] = pltpu.async_copy(
                        rows[k % nbuf], acc_sh.at[didx_v.at[k]],
                        ssem[k % nbuf], add=True)
            for k in (BR - 2, BR - 1):
                gat[k].wait()
                scat[k] = pltpu.async_copy(
                    rows[k % nbuf], acc_sh.at[didx_v.at[k]],
                    ssem[k % nbuf], add=True)
            for k in range(BR - nbuf, BR):
                scat[k].wait()

        plsc.subcore_barrier()
        pltpu.sync_copy(acc_sh.at[pl.ds(s * rpt, rpt)],
                        out_hbm.at[c, pl.ds(s * rpt, rpt)])

    return prop_kernel(src3, dst3, g, zrows)


def _head_body(h_ref, p_ref, degp_ref, Wch_ref, Wf_ref, bf_ref,
               Wx_ref, bx_ref, Wc_ref, bc_ref, y_ref):
    h = h_ref[...]
    dinv = _dinv_of(degp_ref[...])
    p = p_ref[...]
    ph = -dinv * (p[0] + p[1])
    Wch = Wch_ref[...]
    dot = lambda a, b: jnp.dot(a, b, preferred_element_type=jnp.float32)
    f0 = dot(h, Wch[0, 0]) + dot(ph, Wch[0, 1])
    f1 = dot(h, Wch[1, 0]) + dot(ph, Wch[1, 1])
    Wf = Wf_ref[...]
    bf = bf_ref[...]
    hp0 = jnp.tanh(dot(f0, Wf) + bf)
    hp1 = jnp.tanh(dot(f1, Wf) + bf)
    xp = jnp.tanh(dot(h, Wx_ref[...]) + bx_ref[...])
    s0 = jnp.sum(hp0 * xp, axis=1, keepdims=True)
    s1 = jnp.sum(hp1 * xp, axis=1, keepdims=True)
    m = jnp.maximum(s0, s1)
    e0 = jnp.exp(s0 - m)
    e1 = jnp.exp(s1 - m)
    res = (f0 * e0 + f1 * e1) / (e0 + e1)
    y_ref[...] = dot(res, Wc_ref[...]) + bc_ref[...]


def _head(h, part, degp, W_cheb, Wf, bf, Wx, bx, Wc_pad, bc_pad, blk):
    n, h_dim = h.shape
    return pl.pallas_call(
        _head_body,
        grid=(n // blk,),
        in_specs=[
            pl.BlockSpec((blk, h_dim), lambda i: (i, 0)),
            pl.BlockSpec((NC, blk, h_dim), lambda i: (0, i, 0)),
            pl.BlockSpec((NC, blk, 1), lambda i: (0, i, 0)),
            pl.BlockSpec(W_cheb.shape, lambda i: (0, 0, 0, 0)),
            pl.BlockSpec((h_dim, h_dim), lambda i: (0, 0)),
            pl.BlockSpec((1, h_dim), lambda i: (0, 0)),
            pl.BlockSpec((h_dim, h_dim), lambda i: (0, 0)),
            pl.BlockSpec((1, h_dim), lambda i: (0, 0)),
            pl.BlockSpec((h_dim, h_dim), lambda i: (0, 0)),
            pl.BlockSpec((1, h_dim), lambda i: (0, 0)),
        ],
        out_specs=pl.BlockSpec((blk, h_dim), lambda i: (i, 0)),
        out_shape=jax.ShapeDtypeStruct((n, h_dim), jnp.float32),
    )(h, part, degp, W_cheb, Wf, bf.reshape(1, -1),
      Wx, bx.reshape(1, -1), Wc_pad, bc_pad)


def kernel(x, edge_index, W1, b1, W2, b2, W_cheb, Wf, bf, Wx, bx, Wc, bc):
    n, _ = x.shape
    e = edge_index.shape[1]
    h_dim = W1.shape[1]
    c_dim = Wc.shape[1]
    nbat = e // (BR * CH)
    blk = 1024
    n_pad = ((n + blk - 1) // blk) * blk
    rpt = n_pad // NS

    src3 = edge_index[0].reshape(nbat, BR, CH)
    dst3 = edge_index[1].reshape(nbat, BR, CH)
    z128 = jnp.zeros((rpt, h_dim), jnp.float32)
    Wc_pad = jnp.zeros((h_dim, h_dim), jnp.float32).at[:, :c_dim].set(Wc)
    bc_pad = jnp.zeros((1, h_dim), jnp.float32).at[:, :c_dim].set(bc)
    x_pad = jnp.zeros((n_pad, x.shape[1]), jnp.float32).at[:n].set(x)

    deg_flat = _deg_sc(edge_index[0], n_pad)
    degp = deg_flat.reshape(NC, n_pad, 1)
    h, g = _mlpscale(x_pad, W1, b1, W2, b2, degp, blk)
    part = _prop_sc(src3, dst3, g, z128, n_pad)
    y_pad = _head(h, part, degp, W_cheb, Wf, bf, Wx, bx, Wc_pad, bc_pad, blk)
    return y_pad[:n, :c_dim]


# speedup vs baseline: 31.1748x; 1.0900x over previous
import dataclasses
import functools

import jax
import jax.numpy as jnp
from jax import lax
from jax.experimental import pallas as pl
from jax.experimental.pallas import tpu as pltpu
from jax.experimental.pallas import tpu_sc as plsc

NC = 2
NS = 16
NW = NC * NS
CH = 100
BR = 20


def _dinv_of(degp):
    deg = degp[0] + degp[1]
    return jnp.where(deg > 0,
                     lax.rsqrt(jnp.maximum(deg, 1e-12)),
                     jnp.zeros_like(deg))


def _mlpscale_body(x_ref, W1_ref, b1_ref, W2_ref, b2_ref, degp_ref,
                   h_ref, g_ref):
    t = jnp.maximum(
        jnp.dot(x_ref[...], W1_ref[...], preferred_element_type=jnp.float32)
        + b1_ref[...], 0.0)
    h = (jnp.dot(t, W2_ref[...], preferred_element_type=jnp.float32)
         + b2_ref[...])
    h_ref[...] = h
    g_ref[...] = _dinv_of(degp_ref[...]) * h


def _mlpscale(x, W1, b1, W2, b2, degp, blk):
    n, d = x.shape
    h_dim = W1.shape[1]
    out = jax.ShapeDtypeStruct((n, h_dim), jnp.float32)
    return pl.pallas_call(
        _mlpscale_body,
        grid=(n // blk,),
        in_specs=[
            pl.BlockSpec((blk, d), lambda i: (i, 0)),
            pl.BlockSpec((d, h_dim), lambda i: (0, 0)),
            pl.BlockSpec((1, h_dim), lambda i: (0, 0)),
            pl.BlockSpec((h_dim, h_dim), lambda i: (0, 0)),
            pl.BlockSpec((1, h_dim), lambda i: (0, 0)),
            pl.BlockSpec((NC, blk, 1), lambda i: (0, i, 0)),
        ],
        out_specs=[pl.BlockSpec((blk, h_dim), lambda i: (i, 0)),
                   pl.BlockSpec((blk, h_dim), lambda i: (i, 0))],
        out_shape=[out, out],
    )(x, W1, b1.reshape(1, -1), W2, b2.reshape(1, -1), degp)


def _deg_sc(src_flat, n_pad):
    e = src_flat.shape[0]
    ept = e // NW
    rpt = n_pad // NS
    mesh = plsc.VectorSubcoreMesh(core_axis_name="c", subcore_axis_name="s")
    cp = dataclasses.replace(pltpu.CompilerParams(),
                             needs_layout_passes=False)

    @functools.partial(
        pl.kernel,
        out_type=jax.ShapeDtypeStruct((NC * n_pad,), jnp.float32),
        mesh=mesh,
        compiler_params=cp,
        scratch_types=[
            pltpu.VMEM((ept,), jnp.int32),
            pltpu.VMEM((n_pad,), jnp.float32),
            pltpu.VMEM((n_pad,), jnp.float32),
            pltpu.VMEM((rpt,), jnp.float32),
            pltpu.VMEM_SHARED((NS * n_pad,), jnp.float32),
            pltpu.SemaphoreType.DMA,
        ],
    )
    def deg_kernel(src_hbm, out_hbm, idx_v, hist_v, load_v, out_v,
                   stage_sh, sem):
        c = lax.axis_index("c")
        s = lax.axis_index("s")
        w = s * NC + c

        idma = pltpu.async_copy(src_hbm.at[pl.ds(w * ept, ept)], idx_v, sem)

        @pl.loop(0, n_pad, step=16)
        def _(i):
            hist_v[pl.ds(i, 16)] = jnp.zeros((16,), jnp.float32)

        ones16 = jnp.ones((16,), jnp.float32)
        idma.wait()

        @pl.loop(0, ept, step=400)
        def _(b):
            for k in range(25):
                plsc.addupdate_scatter(
                    hist_v, [idx_v[pl.ds(b + k * 16, 16)]], ones16)

        pltpu.sync_copy(hist_v, stage_sh.at[pl.ds(s * n_pad, n_pad)])
        plsc.subcore_barrier()
        descs = [
            pltpu.async_copy(stage_sh.at[pl.ds(r * n_pad + s * rpt, rpt)],
                             load_v.at[pl.ds(r * rpt, rpt)], sem)
            for r in range(NS)
        ]
        for dsc in descs:
            dsc.wait()

        @pl.loop(0, rpt, step=16)
        def _(k):
            t = load_v[pl.ds(k, 16)]
            for r in range(1, NS):
                t = t + load_v[pl.ds(r * rpt + k, 16)]
            out_v[pl.ds(k, 16)] = t

        pltpu.sync_copy(out_v, out_hbm.at[pl.ds(c * n_pad + s * rpt, rpt)])

    return deg_kernel(src_flat)


def _prop_sc(src3, dst3, g, zrows, n_pad):
    h_dim = g.shape[1]
    nbat = src3.shape[0]
    rpt = n_pad // NS
    mesh = plsc.VectorSubcoreMesh(core_axis_name="c", subcore_axis_name="s")

    @functools.partial(
        pl.kernel,
        out_type=jax.ShapeDtypeStruct((NC, n_pad, h_dim), jnp.float32),
        mesh=mesh,
        scratch_types=[
            pltpu.VMEM((BR, CH), jnp.int32),
            pltpu.VMEM((BR, CH), jnp.int32),
        ] + [pltpu.VMEM((CH, h_dim), jnp.float32)] * 3
          + [pltpu.VMEM_SHARED((n_pad, h_dim), jnp.float32)]
          + [pltpu.SemaphoreType.DMA] * 7,
    )
    def prop_kernel(src_hbm, dst_hbm, g_hbm, z_hbm, out_hbm,
                    sidx_v, didx_v, r0, r1, r2, acc_sh,
                    g0, g1, g2, s0, s1, s2, isem):
        rows = (r0, r1, r2)
        gsem = (g0, g1, g2)
        ssem = (s0, s1, s2)
        nbuf = 3
        c = lax.axis_index("c")
        s = lax.axis_index("s")
        w = s * NC + c
        bpt = nbat // NW
        pltpu.sync_copy(z_hbm, acc_sh.at[pl.ds(s * rpt, rpt)])
        plsc.subcore_barrier()

        @pl.loop(w * bpt, (w + 1) * bpt)
        def _(b):
            i0 = pltpu.async_copy(src_hbm.at[b], sidx_v, isem)
            i1 = pltpu.async_copy(dst_hbm.at[b], didx_v, isem)
            i0.wait()
            i1.wait()
            gat = [None] * BR
            scat = [None] * BR
            for j in range(BR):
                bi = j % nbuf
                if j >= nbuf:
                    scat[j - nbuf].wait()
                gat[j] = pltpu.async_copy(
                    g_hbm.at[sidx_v.at[j]], rows[bi], gsem[bi])
                if j >= 2:
                    k = j - 2
                    gat[k].wait()
                    scat[k] = pltpu.async_copy(
                        rows[k % nbuf], acc_sh.at[didx_v.at[k]],
                        ssem[k % nbuf], add=True)
            for k in (BR - 2, BR - 1):
                gat[k].wait()
                scat[k] = pltpu.async_copy(
                    rows[k % nbuf], acc_sh.at[didx_v.at[k]],
                    ssem[k % nbuf], add=True)
            for k in range(BR - nbuf, BR):
                scat[k].wait()

        plsc.subcore_barrier()
        pltpu.sync_copy(acc_sh.at[pl.ds(s * rpt, rpt)],
                        out_hbm.at[c, pl.ds(s * rpt, rpt)])

    return prop_kernel(src3, dst3, g, zrows)


def _head_body(h_ref, p_ref, degp_ref, Wch_ref, Wf_ref, bf_ref,
               Wx_ref, bx_ref, Wc_ref, bc_ref, y_ref):
    h = h_ref[...]
    dinv = _dinv_of(degp_ref[...])
    p = p_ref[...]
    ph = -dinv * (p[0] + p[1])
    Wch = Wch_ref[...]
    dot = lambda a, b: jnp.dot(a, b, preferred_element_type=jnp.float32)
    f0 = dot(h, Wch[0, 0]) + dot(ph, Wch[0, 1])
    f1 = dot(h, Wch[1, 0]) + dot(ph, Wch[1, 1])
    Wf = Wf_ref[...]
    bf = bf_ref[...]
    hp0 = jnp.tanh(dot(f0, Wf) + bf)
    hp1 = jnp.tanh(dot(f1, Wf) + bf)
    xp = jnp.tanh(dot(h, Wx_ref[...]) + bx_ref[...])
    s0 = jnp.sum(hp0 * xp, axis=1, keepdims=True)
    s1 = jnp.sum(hp1 * xp, axis=1, keepdims=True)
    m = jnp.maximum(s0, s1)
    e0 = jnp.exp(s0 - m)
    e1 = jnp.exp(s1 - m)
    res = (f0 * e0 + f1 * e1) / (e0 + e1)
    y_ref[...] = dot(res, Wc_ref[...]) + bc_ref[...]


def _head(h, part, degp, W_cheb, Wf, bf, Wx, bx, Wc_pad, bc_pad, blk):
    n, h_dim = h.shape
    return pl.pallas_call(
        _head_body,
        grid=(n // blk,),
        in_specs=[
            pl.BlockSpec((blk, h_dim), lambda i: (i, 0)),
            pl.BlockSpec((NC, blk, h_dim), lambda i: (0, i, 0)),
            pl.BlockSpec((NC, blk, 1), lambda i: (0, i, 0)),
            pl.BlockSpec(W_cheb.shape, lambda i: (0, 0, 0, 0)),
            pl.BlockSpec((h_dim, h_dim), lambda i: (0, 0)),
            pl.BlockSpec((1, h_dim), lambda i: (0, 0)),
            pl.BlockSpec((h_dim, h_dim), lambda i: (0, 0)),
            pl.BlockSpec((1, h_dim), lambda i: (0, 0)),
            pl.BlockSpec((h_dim, h_dim), lambda i: (0, 0)),
            pl.BlockSpec((1, h_dim), lambda i: (0, 0)),
        ],
        out_specs=pl.BlockSpec((blk, h_dim), lambda i: (i, 0)),
        out_shape=jax.ShapeDtypeStruct((n, h_dim), jnp.float32),
    )(h, part, degp, W_cheb, Wf, bf.reshape(1, -1),
      Wx, bx.reshape(1, -1), Wc_pad, bc_pad)


def kernel(x, edge_index, W1, b1, W2, b2, W_cheb, Wf, bf, Wx, bx, Wc, bc):
    n, _ = x.shape
    e = edge_index.shape[1]
    h_dim = W1.shape[1]
    c_dim = Wc.shape[1]
    nbat = e // (BR * CH)
    blk = 1024
    n_pad = ((n + blk - 1) // blk) * blk
    rpt = n_pad // NS

    src3 = edge_index[0].reshape(nbat, BR, CH)
    dst3 = edge_index[1].reshape(nbat, BR, CH)
    z128 = jnp.zeros((rpt, h_dim), jnp.float32)
    Wc_pad = jnp.zeros((h_dim, h_dim), jnp.float32).at[:, :c_dim].set(Wc)
    bc_pad = jnp.zeros((1, h_dim), jnp.float32).at[:, :c_dim].set(bc)
    x_pad = jnp.zeros((n_pad, x.shape[1]), jnp.float32).at[:n].set(x)

    deg_flat = _deg_sc(edge_index[0], n_pad)
    degp = deg_flat.reshape(NC, n_pad, 1)
    h, g = _mlpscale(x_pad, W1, b1, W2, b2, degp, blk)
    part = _prop_sc(src3, dst3, g, z128, n_pad)
    y_pad = _head(h, part, degp, W_cheb, Wf, bf, Wx, bx, Wc_pad, bc_pad, blk)
    return y_pad[:n, :c_dim]


# speedup vs baseline: 31.1763x; 1.0000x over previous
import dataclasses
import functools

import jax
import jax.numpy as jnp
from jax import lax
from jax.experimental import pallas as pl
from jax.experimental.pallas import tpu as pltpu
from jax.experimental.pallas import tpu_sc as plsc

NC = 2
NS = 16
NW = NC * NS
CH = 100
BR = 20


def _dinv_of(degp):
    deg = degp[0] + degp[1]
    return jnp.where(deg > 0,
                     lax.rsqrt(jnp.maximum(deg, 1e-12)),
                     jnp.zeros_like(deg))


def _mlpscale_body(x_ref, W1_ref, b1_ref, W2_ref, b2_ref, degp_ref,
                   h_ref, g_ref):
    t = jnp.maximum(
        jnp.dot(x_ref[...], W1_ref[...], preferred_element_type=jnp.float32)
        + b1_ref[...], 0.0)
    h = (jnp.dot(t, W2_ref[...], preferred_element_type=jnp.float32)
         + b2_ref[...])
    h_ref[...] = h
    g_ref[...] = _dinv_of(degp_ref[...]) * h


def _mlpscale(x, W1, b1, W2, b2, degp, blk):
    n, d = x.shape
    h_dim = W1.shape[1]
    out = jax.ShapeDtypeStruct((n, h_dim), jnp.float32)
    return pl.pallas_call(
        _mlpscale_body,
        grid=(n // blk,),
        in_specs=[
            pl.BlockSpec((blk, d), lambda i: (i, 0)),
            pl.BlockSpec((d, h_dim), lambda i: (0, 0)),
            pl.BlockSpec((1, h_dim), lambda i: (0, 0)),
            pl.BlockSpec((h_dim, h_dim), lambda i: (0, 0)),
            pl.BlockSpec((1, h_dim), lambda i: (0, 0)),
            pl.BlockSpec((NC, blk, 1), lambda i: (0, i, 0)),
        ],
        out_specs=[pl.BlockSpec((blk, h_dim), lambda i: (i, 0)),
                   pl.BlockSpec((blk, h_dim), lambda i: (i, 0))],
        out_shape=[out, out],
    )(x, W1, b1.reshape(1, -1), W2, b2.reshape(1, -1), degp)


def _deg_sc(src_flat, n_pad):
    e = src_flat.shape[0]
    ept = e // NW
    rpt = n_pad // NS
    mesh = plsc.VectorSubcoreMesh(core_axis_name="c", subcore_axis_name="s")
    cp = dataclasses.replace(pltpu.CompilerParams(),
                             needs_layout_passes=False)

    @functools.partial(
        pl.kernel,
        out_type=jax.ShapeDtypeStruct((NC * n_pad,), jnp.float32),
        mesh=mesh,
        compiler_params=cp,
        scratch_types=[
            pltpu.VMEM((ept,), jnp.int32),
            pltpu.VMEM((n_pad,), jnp.float32),
            pltpu.VMEM((n_pad,), jnp.float32),
            pltpu.VMEM((rpt,), jnp.float32),
            pltpu.VMEM_SHARED((NS * n_pad,), jnp.float32),
            pltpu.SemaphoreType.DMA,
        ],
    )
    def deg_kernel(src_hbm, out_hbm, idx_v, hist_v, load_v, out_v,
                   stage_sh, sem):
        c = lax.axis_index("c")
        s = lax.axis_index("s")
        w = s * NC + c

        idma = pltpu.async_copy(src_hbm.at[pl.ds(w * ept, ept)], idx_v, sem)

        @pl.loop(0, n_pad, step=16)
        def _(i):
            hist_v[pl.ds(i, 16)] = jnp.zeros((16,), jnp.float32)

        ones16 = jnp.ones((16,), jnp.float32)
        idma.wait()

        @pl.loop(0, ept, step=400)
        def _(b):
            for k in range(25):
                plsc.addupdate_scatter(
                    hist_v, [idx_v[pl.ds(b + k * 16, 16)]], ones16)

        pltpu.sync_copy(hist_v, stage_sh.at[pl.ds(s * n_pad, n_pad)])
        plsc.subcore_barrier()
        descs = [
            pltpu.async_copy(stage_sh.at[pl.ds(r * n_pad + s * rpt, rpt)],
                             load_v.at[pl.ds(r * rpt, rpt)], sem)
            for r in range(NS)
        ]
        for dsc in descs:
            dsc.wait()

        @pl.loop(0, rpt, step=16)
        def _(k):
            t = load_v[pl.ds(k, 16)]
            for r in range(1, NS):
                t = t + load_v[pl.ds(r * rpt + k, 16)]
            out_v[pl.ds(k, 16)] = t

        pltpu.sync_copy(out_v, out_hbm.at[pl.ds(c * n_pad + s * rpt, rpt)])

    return deg_kernel(src_flat)


def _prop_sc(src3, dst3, g, zrows, n_pad):
    h_dim = g.shape[1]
    nbat = src3.shape[0]
    rpt = n_pad // NS
    mesh = plsc.VectorSubcoreMesh(core_axis_name="c", subcore_axis_name="s")

    @functools.partial(
        pl.kernel,
        out_type=jax.ShapeDtypeStruct((NC, n_pad, h_dim), jnp.float32),
        mesh=mesh,
        scratch_types=[
            pltpu.VMEM((BR, CH), jnp.int32),
            pltpu.VMEM((BR, CH), jnp.int32),
        ] + [pltpu.VMEM((CH, h_dim), jnp.float32)] * 3
          + [pltpu.VMEM_SHARED((n_pad, h_dim), jnp.float32)]
          + [pltpu.SemaphoreType.DMA] * 7,
    )
    def prop_kernel(src_hbm, dst_hbm, g_hbm, z_hbm, out_hbm,
                    sidx_v, didx_v, r0, r1, r2, acc_sh,
                    g0, g1, g2, s0, s1, s2, isem):
        rows = (r0, r1, r2)
        gsem = (g0, g1, g2)
        ssem = (s0, s1, s2)
        nbuf = 3
        c = lax.axis_index("c")
        s = lax.axis_index("s")
        w = s * NC + c
        bpt = nbat // NW
        pltpu.sync_copy(z_hbm, acc_sh.at[pl.ds(s * rpt, rpt)])
        plsc.subcore_barrier()

        @pl.loop(w * bpt, (w + 1) * bpt)
        def _(b):
            i0 = pltpu.async_copy(src_hbm.at[b], sidx_v, isem)
            i1 = pltpu.async_copy(dst_hbm.at[b], didx_v, isem)
            i0.wait()
            i1.wait()
            gat = [None] * BR
            scat = [None] * BR
            for j in range(BR):
                bi = j % nbuf
                if j >= nbuf:
                    scat[j - nbuf].wait()
                gat[j] = pltpu.async_copy(
                    g_hbm.at[sidx_v.at[j]], rows[bi], gsem[bi])
                if j >= 2:
                    k = j - 2
                    gat[k].wait()
                    scat[k] = pltpu.async_copy(
                        rows[k % nbuf], acc_sh.at[didx_v.at[k]],
                        ssem[k % nbuf], add=True)
            for k in (BR - 2, BR - 1):
                gat[k].wait()
                scat[k] = pltpu.async_copy(
                    rows[k % nbuf], acc_sh.at[didx_v.at[k]],
                    ssem[k % nbuf], add=True)
            for k in range(BR - nbuf, BR):
                scat[k].wait()

        plsc.subcore_barrier()
        pltpu.sync_copy(acc_sh.at[pl.ds(s * rpt, rpt)],
                        out_hbm.at[c, pl.ds(s * rpt, rpt)])

    return prop_kernel(src3, dst3, g, zrows)


def _head_body(h_ref, p_ref, degp_ref, Wch_ref, Wf_ref, bf_ref,
               Wx_ref, bx_ref, Wc_ref, bc_ref, y_ref):
    h = h_ref[...]
    dinv = _dinv_of(degp_ref[...])
    p = p_ref[...]
    ph = -dinv * (p[0] + p[1])
    Wch = Wch_ref[...]
    dot = lambda a, b: jnp.dot(a, b, preferred_element_type=jnp.float32)
    f0 = dot(h, Wch[0, 0]) + dot(ph, Wch[0, 1])
    f1 = dot(h, Wch[1, 0]) + dot(ph, Wch[1, 1])
    Wf = Wf_ref[...]
    bf = bf_ref[...]
    hp0 = jnp.tanh(dot(f0, Wf) + bf)
    hp1 = jnp.tanh(dot(f1, Wf) + bf)
    xp = jnp.tanh(dot(h, Wx_ref[...]) + bx_ref[...])
    s0 = jnp.sum(hp0 * xp, axis=1, keepdims=True)
    s1 = jnp.sum(hp1 * xp, axis=1, keepdims=True)
    m = jnp.maximum(s0, s1)
    e0 = jnp.exp(s0 - m)
    e1 = jnp.exp(s1 - m)
    res = (f0 * e0 + f1 * e1) / (e0 + e1)
    y_ref[...] = dot(res, Wc_ref[...]) + bc_ref[...]


def _head(h, part, degp, W_cheb, Wf, bf, Wx, bx, Wc_pad, bc_pad, blk):
    n, h_dim = h.shape
    return pl.pallas_call(
        _head_body,
        grid=(n // blk,),
        in_specs=[
            pl.BlockSpec((blk, h_dim), lambda i: (i, 0)),
            pl.BlockSpec((NC, blk, h_dim), lambda i: (0, i, 0)),
            pl.BlockSpec((NC, blk, 1), lambda i: (0, i, 0)),
            pl.BlockSpec(W_cheb.shape, lambda i: (0, 0, 0, 0)),
            pl.BlockSpec((h_dim, h_dim), lambda i: (0, 0)),
            pl.BlockSpec((1, h_dim), lambda i: (0, 0)),
            pl.BlockSpec((h_dim, h_dim), lambda i: (0, 0)),
            pl.BlockSpec((1, h_dim), lambda i: (0, 0)),
            pl.BlockSpec((h_dim, 8), lambda i: (0, 0)),
            pl.BlockSpec((1, 8), lambda i: (0, 0)),
        ],
        out_specs=pl.BlockSpec((blk, 8), lambda i: (i, 0)),
        out_shape=jax.ShapeDtypeStruct((n, 8), jnp.float32),
    )(h, part, degp, W_cheb, Wf, bf.reshape(1, -1),
      Wx, bx.reshape(1, -1), Wc_pad, bc_pad)


def kernel(x, edge_index, W1, b1, W2, b2, W_cheb, Wf, bf, Wx, bx, Wc, bc):
    n, _ = x.shape
    e = edge_index.shape[1]
    h_dim = W1.shape[1]
    c_dim = Wc.shape[1]
    nbat = e // (BR * CH)
    blk = 1024
    n_pad = ((n + blk - 1) // blk) * blk
    rpt = n_pad // NS

    src3 = edge_index[0].reshape(nbat, BR, CH)
    dst3 = edge_index[1].reshape(nbat, BR, CH)
    z128 = jnp.zeros((rpt, h_dim), jnp.float32)
    Wc_pad = jnp.zeros((h_dim, 8), jnp.float32).at[:, :c_dim].set(Wc)
    bc_pad = jnp.zeros((1, 8), jnp.float32).at[:, :c_dim].set(bc)
    x_pad = jnp.zeros((n_pad, x.shape[1]), jnp.float32).at[:n].set(x)

    deg_flat = _deg_sc(edge_index[0], n_pad)
    degp = deg_flat.reshape(NC, n_pad, 1)
    h, g = _mlpscale(x_pad, W1, b1, W2, b2, degp, blk)
    part = _prop_sc(src3, dst3, g, z128, n_pad)
    y_pad = _head(h, part, degp, W_cheb, Wf, bf, Wx, bx, Wc_pad, bc_pad, blk)
    return y_pad[:n, :c_dim]


# speedup vs baseline: 31.5471x; 1.0119x over previous
import dataclasses
import functools

import jax
import jax.numpy as jnp
from jax import lax
from jax.experimental import pallas as pl
from jax.experimental.pallas import tpu as pltpu
from jax.experimental.pallas import tpu_sc as plsc

NC = 2
NS = 16
NW = NC * NS
CH = 100
BR = 25


def _dinv_of(degp):
    deg = degp[0] + degp[1]
    return jnp.where(deg > 0,
                     lax.rsqrt(jnp.maximum(deg, 1e-12)),
                     jnp.zeros_like(deg))


def _mlpscale_body(x_ref, W1_ref, b1_ref, W2_ref, b2_ref, degp_ref,
                   h_ref, g_ref):
    t = jnp.maximum(
        jnp.dot(x_ref[...], W1_ref[...], preferred_element_type=jnp.float32)
        + b1_ref[...], 0.0)
    h = (jnp.dot(t, W2_ref[...], preferred_element_type=jnp.float32)
         + b2_ref[...])
    h_ref[...] = h
    g_ref[...] = _dinv_of(degp_ref[...]) * h


def _mlpscale(x, W1, b1, W2, b2, degp, blk):
    n, d = x.shape
    h_dim = W1.shape[1]
    out = jax.ShapeDtypeStruct((n, h_dim), jnp.float32)
    return pl.pallas_call(
        _mlpscale_body,
        grid=(n // blk,),
        in_specs=[
            pl.BlockSpec((blk, d), lambda i: (i, 0)),
            pl.BlockSpec((d, h_dim), lambda i: (0, 0)),
            pl.BlockSpec((1, h_dim), lambda i: (0, 0)),
            pl.BlockSpec((h_dim, h_dim), lambda i: (0, 0)),
            pl.BlockSpec((1, h_dim), lambda i: (0, 0)),
            pl.BlockSpec((NC, blk, 1), lambda i: (0, i, 0)),
        ],
        out_specs=[pl.BlockSpec((blk, h_dim), lambda i: (i, 0)),
                   pl.BlockSpec((blk, h_dim), lambda i: (i, 0))],
        out_shape=[out, out],
    )(x, W1, b1.reshape(1, -1), W2, b2.reshape(1, -1), degp)


def _deg_sc(src_flat, n_pad):
    e = src_flat.shape[0]
    ept = e // NW
    rpt = n_pad // NS
    mesh = plsc.VectorSubcoreMesh(core_axis_name="c", subcore_axis_name="s")
    cp = dataclasses.replace(pltpu.CompilerParams(),
                             needs_layout_passes=False)

    @functools.partial(
        pl.kernel,
        out_type=jax.ShapeDtypeStruct((NC * n_pad,), jnp.float32),
        mesh=mesh,
        compiler_params=cp,
        scratch_types=[
            pltpu.VMEM((ept,), jnp.int32),
            pltpu.VMEM((n_pad,), jnp.float32),
            pltpu.VMEM((n_pad,), jnp.float32),
            pltpu.VMEM((rpt,), jnp.float32),
            pltpu.VMEM_SHARED((NS * n_pad,), jnp.float32),
            pltpu.SemaphoreType.DMA,
        ],
    )
    def deg_kernel(src_hbm, out_hbm, idx_v, hist_v, load_v, out_v,
                   stage_sh, sem):
        c = lax.axis_index("c")
        s = lax.axis_index("s")
        w = s * NC + c

        idma = pltpu.async_copy(src_hbm.at[pl.ds(w * ept, ept)], idx_v, sem)

        @pl.loop(0, n_pad, step=16)
        def _(i):
            hist_v[pl.ds(i, 16)] = jnp.zeros((16,), jnp.float32)

        ones16 = jnp.ones((16,), jnp.float32)
        idma.wait()

        @pl.loop(0, ept, step=400)
        def _(b):
            for k in range(25):
                plsc.addupdate_scatter(
                    hist_v, [idx_v[pl.ds(b + k * 16, 16)]], ones16)

        pltpu.sync_copy(hist_v, stage_sh.at[pl.ds(s * n_pad, n_pad)])
        plsc.subcore_barrier()
        descs = [
            pltpu.async_copy(stage_sh.at[pl.ds(r * n_pad + s * rpt, rpt)],
                             load_v.at[pl.ds(r * rpt, rpt)], sem)
            for r in range(NS)
        ]
        for dsc in descs:
            dsc.wait()

        @pl.loop(0, rpt, step=16)
        def _(k):
            t = load_v[pl.ds(k, 16)]
            for r in range(1, NS):
                t = t + load_v[pl.ds(r * rpt + k, 16)]
            out_v[pl.ds(k, 16)] = t

        pltpu.sync_copy(out_v, out_hbm.at[pl.ds(c * n_pad + s * rpt, rpt)])

    return deg_kernel(src_flat)


def _prop_sc(src3, dst3, g, zrows, n_pad):
    h_dim = g.shape[1]
    nbat = src3.shape[0]
    rpt = n_pad // NS
    mesh = plsc.VectorSubcoreMesh(core_axis_name="c", subcore_axis_name="s")

    @functools.partial(
        pl.kernel,
        out_type=jax.ShapeDtypeStruct((NC, n_pad, h_dim), jnp.float32),
        mesh=mesh,
        scratch_types=[
            pltpu.VMEM((BR, CH), jnp.int32),
            pltpu.VMEM((BR, CH), jnp.int32),
        ] + [pltpu.VMEM((CH, h_dim), jnp.float32)] * 3
          + [pltpu.VMEM_SHARED((n_pad, h_dim), jnp.float32)]
          + [pltpu.SemaphoreType.DMA] * 7,
    )
    def prop_kernel(src_hbm, dst_hbm, g_hbm, z_hbm, out_hbm,
                    sidx_v, didx_v, r0, r1, r2, acc_sh,
                    g0, g1, g2, s0, s1, s2, isem):
        rows = (r0, r1, r2)
        gsem = (g0, g1, g2)
        ssem = (s0, s1, s2)
        nbuf = 3
        c = lax.axis_index("c")
        s = lax.axis_index("s")
        w = s * NC + c
        bpt = nbat // NW
        pltpu.sync_copy(z_hbm, acc_sh.at[pl.ds(s * rpt, rpt)])
        plsc.subcore_barrier()

        @pl.loop(w * bpt, (w + 1) * bpt)
        def _(b):
            i0 = pltpu.async_copy(src_hbm.at[b], sidx_v, isem)
            i1 = pltpu.async_copy(dst_hbm.at[b], didx_v, isem)
            i0.wait()
            i1.wait()
            gat = [None] * BR
            scat = [None] * BR
            for j in range(BR):
                bi = j % nbuf
                if j >= nbuf:
                    scat[j - nbuf].wait()
                gat[j] = pltpu.async_copy(
                    g_hbm.at[sidx_v.at[j]], rows[bi], gsem[bi])
                if j >= 2:
                    k = j - 2
                    gat[k].wait()
                    scat[k] = pltpu.async_copy(
                        rows[k % nbuf], acc_sh.at[didx_v.at[k]],
                        ssem[k % nbuf], add=True)
            for k in (BR - 2, BR - 1):
                gat[k].wait()
                scat[k] = pltpu.async_copy(
                    rows[k % nbuf], acc_sh.at[didx_v.at[k]],
                    ssem[k % nbuf], add=True)
            for k in range(BR - nbuf, BR):
                scat[k].wait()

        plsc.subcore_barrier()
        pltpu.sync_copy(acc_sh.at[pl.ds(s * rpt, rpt)],
                        out_hbm.at[c, pl.ds(s * rpt, rpt)])

    return prop_kernel(src3, dst3, g, zrows)


def _head_body(h_ref, p_ref, degp_ref, Wch_ref, Wf_ref, bf_ref,
               Wx_ref, bx_ref, Wc_ref, bc_ref, y_ref):
    h = h_ref[...]
    dinv = _dinv_of(degp_ref[...])
    p = p_ref[...]
    ph = -dinv * (p[0] + p[1])
    Wch = Wch_ref[...]
    dot = lambda a, b: jnp.dot(a, b, preferred_element_type=jnp.float32)
    f0 = dot(h, Wch[0, 0]) + dot(ph, Wch[0, 1])
    f1 = dot(h, Wch[1, 0]) + dot(ph, Wch[1, 1])
    Wf = Wf_ref[...]
    bf = bf_ref[...]
    hp0 = jnp.tanh(dot(f0, Wf) + bf)
    hp1 = jnp.tanh(dot(f1, Wf) + bf)
    xp = jnp.tanh(dot(h, Wx_ref[...]) + bx_ref[...])
    s0 = jnp.sum(hp0 * xp, axis=1, keepdims=True)
    s1 = jnp.sum(hp1 * xp, axis=1, keepdims=True)
    m = jnp.maximum(s0, s1)
    e0 = jnp.exp(s0 - m)
    e1 = jnp.exp(s1 - m)
    res = (f0 * e0 + f1 * e1) / (e0 + e1)
    y_ref[...] = dot(res, Wc_ref[...]) + bc_ref[...]


def _head(h, part, degp, W_cheb, Wf, bf, Wx, bx, Wc_pad, bc_pad, blk):
    n, h_dim = h.shape
    return pl.pallas_call(
        _head_body,
        grid=(n // blk,),
        in_specs=[
            pl.BlockSpec((blk, h_dim), lambda i: (i, 0)),
            pl.BlockSpec((NC, blk, h_dim), lambda i: (0, i, 0)),
            pl.BlockSpec((NC, blk, 1), lambda i: (0, i, 0)),
            pl.BlockSpec(W_cheb.shape, lambda i: (0, 0, 0, 0)),
            pl.BlockSpec((h_dim, h_dim), lambda i: (0, 0)),
            pl.BlockSpec((1, h_dim), lambda i: (0, 0)),
            pl.BlockSpec((h_dim, h_dim), lambda i: (0, 0)),
            pl.BlockSpec((1, h_dim), lambda i: (0, 0)),
            pl.BlockSpec((h_dim, 8), lambda i: (0, 0)),
            pl.BlockSpec((1, 8), lambda i: (0, 0)),
        ],
        out_specs=pl.BlockSpec((blk, 8), lambda i: (i, 0)),
        out_shape=jax.ShapeDtypeStruct((n, 8), jnp.float32),
    )(h, part, degp, W_cheb, Wf, bf.reshape(1, -1),
      Wx, bx.reshape(1, -1), Wc_pad, bc_pad)


def kernel(x, edge_index, W1, b1, W2, b2, W_cheb, Wf, bf, Wx, bx, Wc, bc):
    n, _ = x.shape
    e = edge_index.shape[1]
    h_dim = W1.shape[1]
    c_dim = Wc.shape[1]
    nbat = e // (BR * CH)
    blk = 1024
    n_pad = ((n + blk - 1) // blk) * blk
    rpt = n_pad // NS

    src3 = edge_index[0].reshape(nbat, BR, CH)
    dst3 = edge_index[1].reshape(nbat, BR, CH)
    z128 = jnp.zeros((rpt, h_dim), jnp.float32)
    Wc_pad = jnp.zeros((h_dim, 8), jnp.float32).at[:, :c_dim].set(Wc)
    bc_pad = jnp.zeros((1, 8), jnp.float32).at[:, :c_dim].set(bc)
    x_pad = jnp.zeros((n_pad, x.shape[1]), jnp.float32).at[:n].set(x)

    deg_flat = _deg_sc(edge_index[0], n_pad)
    degp = deg_flat.reshape(NC, n_pad, 1)
    h, g = _mlpscale(x_pad, W1, b1, W2, b2, degp, blk)
    part = _prop_sc(src3, dst3, g, z128, n_pad)
    y_pad = _head(h, part, degp, W_cheb, Wf, bf, Wx, bx, Wc_pad, bc_pad, blk)
    return y_pad[:n, :c_dim]
